# Initial kernel scaffold; baseline (speedup 1.0000x reference)
import jax
import jax.numpy as jnp
from jax.experimental import pallas as pl


def kernel(h, edge_index, e, W_A1, b_A1, W_A2, b_A2, W_B1, b_B1, W_B2, b_B2, W_B3, b_B3, g_h, be_h, g_e, be_e):
    raise NotImplementedError("write your pallas kernel here")


# speedup vs baseline: 2.4044x; 2.4044x over previous
import functools

import jax
import jax.numpy as jnp
from jax import lax
from jax.experimental import pallas as pl
from jax.experimental.pallas import tpu as pltpu
from jax.experimental.pallas import tpu_sc as plsc

N = 10000
E = 320000
D = 128

_NC = 2
_NS = 16
_NW = _NC * _NS

_C = 80
_NB = E // _C
_LANES = 16
_VPR = D // _LANES

_BN = 2000
_BE = 4000


def _mm4_body(h_ref, w1_ref, w2_ref, w3_ref, w4_ref, b_ref,
              o1_ref, o2_ref, o3_ref, o4_ref):
    hb = h_ref[...]
    o1_ref[...] = jnp.dot(hb, w1_ref[...], preferred_element_type=jnp.float32) + b_ref[0:1, :]
    o2_ref[...] = jnp.dot(hb, w2_ref[...], preferred_element_type=jnp.float32) + b_ref[1:2, :]
    o3_ref[...] = jnp.dot(hb, w3_ref[...], preferred_element_type=jnp.float32) + b_ref[2:3, :]
    o4_ref[...] = jnp.dot(hb, w4_ref[...], preferred_element_type=jnp.float32) + b_ref[3:4, :]


def _edge_body(e_ref, bsum_ref, wt_ref, vec_ref, out_ref):
    eb = e_ref[...]
    x = jnp.dot(eb, wt_ref[...], preferred_element_type=jnp.float32)
    x = x + bsum_ref[...] + vec_ref[0:1, :]
    mu = jnp.mean(x, axis=1, keepdims=True)
    xc = x - mu
    var = jnp.mean(xc * xc, axis=1, keepdims=True)
    y = xc * lax.rsqrt(var + 1e-5)
    y = y * vec_ref[1:2, :] + vec_ref[2:3, :]
    out_ref[...] = jnp.maximum(y, 0.0) + eb


def _final_body(a1_ref, sh_ref, ss_ref, h_ref, vec_ref, out_ref):
    x = a1_ref[...] + sh_ref[...] / (ss_ref[...] + 1e-6)
    mu = jnp.mean(x, axis=1, keepdims=True)
    xc = x - mu
    var = jnp.mean(xc * xc, axis=1, keepdims=True)
    y = xc * lax.rsqrt(var + 1e-5)
    y = y * vec_ref[0:1, :] + vec_ref[1:2, :]
    out_ref[...] = jnp.maximum(y, 0.0) + h_ref[...]


def _node_spec(i):
    return (i, 0)


def _rep_spec(i):
    return (0, 0)


def _mm4(h, wt1, wt2, wt3, wt4, nbias):
    grid = (N // _BN,)
    blk = pl.BlockSpec((_BN, D), _node_spec)
    wspec = pl.BlockSpec((D, D), _rep_spec)
    return pl.pallas_call(
        _mm4_body,
        grid=grid,
        in_specs=[blk, wspec, wspec, wspec, wspec,
                  pl.BlockSpec((8, D), _rep_spec)],
        out_specs=[blk, blk, blk, blk],
        out_shape=[jax.ShapeDtypeStruct((N, D), jnp.float32)] * 4,
    )(h, wt1, wt2, wt3, wt4, nbias)


def _edge_fused(e, bsum, wt3, evec):
    grid = (E // _BE,)
    blk = pl.BlockSpec((_BE, D), _node_spec)
    return pl.pallas_call(
        _edge_body,
        grid=grid,
        in_specs=[blk, blk, pl.BlockSpec((D, D), _rep_spec),
                  pl.BlockSpec((8, D), _rep_spec)],
        out_specs=blk,
        out_shape=jax.ShapeDtypeStruct((E, D), jnp.float32),
    )(e, bsum, wt3, evec)


def _node_final(a1h, sum_h, sum_sig, h, hvec):
    grid = (N // _BN,)
    blk = pl.BlockSpec((_BN, D), _node_spec)
    return pl.pallas_call(
        _final_body,
        grid=grid,
        in_specs=[blk, blk, blk, blk, pl.BlockSpec((8, D), _rep_spec)],
        out_specs=blk,
        out_shape=jax.ShapeDtypeStruct((N, D), jnp.float32),
    )(a1h, sum_h, sum_sig, h, hvec)


def _gather_sum_body(b1_hbm, b2_hbm, src_hbm, dst_hbm, out_hbm,
                     idx_s, idx_d, g1, g2, sem0, sem1):
    cid = lax.axis_index("c")
    sid = lax.axis_index("s")
    w = sid * _NC + cid

    def chunk(i, carry):
        start = (i * _NW + w) * _C
        pltpu.sync_copy(src_hbm.at[pl.ds(start, _C)], idx_s)
        pltpu.sync_copy(dst_hbm.at[pl.ds(start, _C)], idx_d)
        cp1 = pltpu.async_copy(b1_hbm.at[idx_s], g1, sem0)
        cp2 = pltpu.async_copy(b2_hbm.at[idx_d], g2, sem1)
        cp1.wait()
        cp2.wait()

        def row(r, c2):
            for j in range(_VPR):
                sl = pl.ds(j * _LANES, _LANES)
                g1[r, sl] = g1[r, sl] + g2[r, sl]
            return c2

        lax.fori_loop(0, _C, row, 0)
        pltpu.sync_copy(g1, out_hbm.at[pl.ds(start, _C)])
        return carry

    lax.fori_loop(0, _NB // _NW, chunk, 0)


def _scatter_body(eji_hbm, src_hbm, dst_hbm, a2_hbm, zeros_hbm,
                  out_sig, out_h,
                  idx_s, idx_d, buf, a2buf, acc, sem0, sem1):
    cid = lax.axis_index("c")
    sid = lax.axis_index("s")

    @pl.when(sid == 0)
    def _():
        pltpu.sync_copy(zeros_hbm, acc)

    plsc.subcore_barrier()

    def chunk(i, carry):
        start = (i * _NS + sid) * _C
        pltpu.sync_copy(dst_hbm.at[pl.ds(start, _C)], idx_d)
        pltpu.sync_copy(eji_hbm.at[pl.ds(start, _C)], buf)

        @pl.when(cid == 0)
        def _():
            def row(r, c2):
                for j in range(_VPR):
                    sl = pl.ds(j * _LANES, _LANES)
                    x = buf[r, sl]
                    buf[r, sl] = 1.0 / (1.0 + jnp.exp(-x))
                return c2
            lax.fori_loop(0, _C, row, 0)

        @pl.when(cid == 1)
        def _():
            pltpu.sync_copy(src_hbm.at[pl.ds(start, _C)], idx_s)
            pltpu.async_copy(a2_hbm.at[idx_s], a2buf, sem1).wait()

            def row(r, c2):
                for j in range(_VPR):
                    sl = pl.ds(j * _LANES, _LANES)
                    x = buf[r, sl]
                    s = 1.0 / (1.0 + jnp.exp(-x))
                    buf[r, sl] = s * a2buf[r, sl]
                return c2
            lax.fori_loop(0, _C, row, 0)

        pltpu.sync_copy(buf, acc.at[idx_d], add=True)
        return carry

    lax.fori_loop(0, _NB // _NS, chunk, 0)

    plsc.subcore_barrier()

    @pl.when((sid == 0) & (cid == 0))
    def _():
        pltpu.sync_copy(acc, out_sig)

    @pl.when((sid == 0) & (cid == 1))
    def _():
        pltpu.sync_copy(acc, out_h)


def _sc_mesh():
    return plsc.VectorSubcoreMesh(core_axis_name="c", subcore_axis_name="s",
                                  num_cores=_NC, num_subcores=_NS)


def _gather_sum(b1h, b2h, src, dst):
    return pl.kernel(
        _gather_sum_body,
        out_type=jax.ShapeDtypeStruct((E, D), jnp.float32),
        mesh=_sc_mesh(),
        scratch_types=[
            pltpu.VMEM((_C,), jnp.int32),
            pltpu.VMEM((_C,), jnp.int32),
            pltpu.VMEM((_C, D), jnp.float32),
            pltpu.VMEM((_C, D), jnp.float32),
            pltpu.SemaphoreType.DMA,
            pltpu.SemaphoreType.DMA,
        ],
    )(b1h, b2h, src, dst)


def _scatter_sums(e_ji, src, dst, a2h, zeros_nd):
    return pl.kernel(
        _scatter_body,
        out_type=(jax.ShapeDtypeStruct((N, D), jnp.float32),
                  jax.ShapeDtypeStruct((N, D), jnp.float32)),
        mesh=_sc_mesh(),
        scratch_types=[
            pltpu.VMEM((_C,), jnp.int32),
            pltpu.VMEM((_C,), jnp.int32),
            pltpu.VMEM((_C, D), jnp.float32),
            pltpu.VMEM((_C, D), jnp.float32),
            pltpu.VMEM_SHARED((N, D), jnp.float32),
            pltpu.SemaphoreType.DMA,
            pltpu.SemaphoreType.DMA,
        ],
    )(e_ji, src, dst, a2h, zeros_nd)


def kernel(h, edge_index, e, W_A1, b_A1, W_A2, b_A2, W_B1, b_B1,
           W_B2, b_B2, W_B3, b_B3, g_h, be_h, g_e, be_e):
    src = edge_index[0]
    dst = edge_index[1]

    zpad = jnp.zeros((4, D), jnp.float32)
    nbias = jnp.concatenate([b_A1[None], b_A2[None], b_B1[None], b_B2[None],
                             zpad], axis=0)
    evec = jnp.concatenate([b_B3[None], g_e[None], be_e[None], zpad,
                            jnp.zeros((1, D), jnp.float32)], axis=0)
    hvec = jnp.concatenate([g_h[None], be_h[None], zpad,
                            jnp.zeros((2, D), jnp.float32)], axis=0)

    a1h, a2h, b1h, b2h = _mm4(h, W_A1.T, W_A2.T, W_B1.T, W_B2.T, nbias)
    bsum = _gather_sum(b1h, b2h, src, dst)
    e_ji = _edge_fused(e, bsum, W_B3.T, evec)
    zeros_nd = jnp.zeros((N, D), jnp.float32)
    sum_sig, sum_h = _scatter_sums(e_ji, src, dst, a2h, zeros_nd)
    h_out = _node_final(a1h, sum_h, sum_sig, h, hvec)
    return (h_out, e_ji)


# speedup vs baseline: 2.6181x; 1.0889x over previous
import functools

import jax
import jax.numpy as jnp
from jax import lax
from jax.experimental import pallas as pl
from jax.experimental.pallas import tpu as pltpu
from jax.experimental.pallas import tpu_sc as plsc

N = 10000
E = 320000
D = 128

_NC = 2
_NS = 16
_NW = _NC * _NS

_C = 80
_NB = E // _C
_LANES = 16
_VPR = D // _LANES

_BN = 2000
_BE = 4000


def _mm4_body(h_ref, w1_ref, w2_ref, w3_ref, w4_ref, b_ref,
              o1_ref, o2_ref, o3_ref, o4_ref):
    hb = h_ref[...]
    o1_ref[...] = jnp.dot(hb, w1_ref[...], preferred_element_type=jnp.float32) + b_ref[0:1, :]
    o2_ref[...] = jnp.dot(hb, w2_ref[...], preferred_element_type=jnp.float32) + b_ref[1:2, :]
    o3_ref[...] = jnp.dot(hb, w3_ref[...], preferred_element_type=jnp.float32) + b_ref[2:3, :]
    o4_ref[...] = jnp.dot(hb, w4_ref[...], preferred_element_type=jnp.float32) + b_ref[3:4, :]


def _edge_body(e_ref, bsum_ref, wt_ref, vec_ref, out_ref, sig_ref):
    eb = e_ref[...]
    x = jnp.dot(eb, wt_ref[...], preferred_element_type=jnp.float32)
    x = x + bsum_ref[...] + vec_ref[0:1, :]
    mu = jnp.mean(x, axis=1, keepdims=True)
    xc = x - mu
    var = jnp.mean(xc * xc, axis=1, keepdims=True)
    y = xc * lax.rsqrt(var + 1e-5)
    y = y * vec_ref[1:2, :] + vec_ref[2:3, :]
    e_ji = jnp.maximum(y, 0.0) + eb
    out_ref[...] = e_ji
    sig_ref[...] = jax.nn.sigmoid(e_ji)


def _final_body(a1_ref, sh_ref, ss_ref, h_ref, vec_ref, out_ref):
    x = a1_ref[...] + sh_ref[...] / (ss_ref[...] + 1e-6)
    mu = jnp.mean(x, axis=1, keepdims=True)
    xc = x - mu
    var = jnp.mean(xc * xc, axis=1, keepdims=True)
    y = xc * lax.rsqrt(var + 1e-5)
    y = y * vec_ref[0:1, :] + vec_ref[1:2, :]
    out_ref[...] = jnp.maximum(y, 0.0) + h_ref[...]


def _node_spec(i):
    return (i, 0)


def _rep_spec(i):
    return (0, 0)


def _mm4(h, wt1, wt2, wt3, wt4, nbias):
    grid = (N // _BN,)
    blk = pl.BlockSpec((_BN, D), _node_spec)
    wspec = pl.BlockSpec((D, D), _rep_spec)
    return pl.pallas_call(
        _mm4_body,
        grid=grid,
        in_specs=[blk, wspec, wspec, wspec, wspec,
                  pl.BlockSpec((8, D), _rep_spec)],
        out_specs=[blk, blk, blk, blk],
        out_shape=[jax.ShapeDtypeStruct((N, D), jnp.float32)] * 4,
    )(h, wt1, wt2, wt3, wt4, nbias)


def _edge_fused(e, bsum, wt3, evec):
    grid = (E // _BE,)
    blk = pl.BlockSpec((_BE, D), _node_spec)
    return pl.pallas_call(
        _edge_body,
        grid=grid,
        in_specs=[blk, blk, pl.BlockSpec((D, D), _rep_spec),
                  pl.BlockSpec((8, D), _rep_spec)],
        out_specs=[blk, blk],
        out_shape=[jax.ShapeDtypeStruct((E, D), jnp.float32)] * 2,
    )(e, bsum, wt3, evec)


def _node_final(a1h, sum_h, sum_sig, h, hvec):
    grid = (N // _BN,)
    blk = pl.BlockSpec((_BN, D), _node_spec)
    return pl.pallas_call(
        _final_body,
        grid=grid,
        in_specs=[blk, blk, blk, blk, pl.BlockSpec((8, D), _rep_spec)],
        out_specs=blk,
        out_shape=jax.ShapeDtypeStruct((N, D), jnp.float32),
    )(a1h, sum_h, sum_sig, h, hvec)


def _gather_sum_body(b1_hbm, b2_hbm, src_hbm, dst_hbm, out_hbm,
                     idx_s, idx_d, g1, g2, sem0, sem1):
    cid = lax.axis_index("c")
    sid = lax.axis_index("s")
    w = sid * _NC + cid

    def chunk(i, carry):
        start = (i * _NW + w) * _C
        pltpu.sync_copy(src_hbm.at[pl.ds(start, _C)], idx_s)
        pltpu.sync_copy(dst_hbm.at[pl.ds(start, _C)], idx_d)
        cp1 = pltpu.async_copy(b1_hbm.at[idx_s], g1, sem0)
        cp2 = pltpu.async_copy(b2_hbm.at[idx_d], g2, sem1)
        cp1.wait()
        cp2.wait()

        def row(r, c2):
            for j in range(_VPR):
                sl = pl.ds(j * _LANES, _LANES)
                g1[r, sl] = g1[r, sl] + g2[r, sl]
            return c2

        lax.fori_loop(0, _C, row, 0)
        pltpu.sync_copy(g1, out_hbm.at[pl.ds(start, _C)])
        return carry

    lax.fori_loop(0, _NB // _NW, chunk, 0)


def _scatter_body(sig_hbm, src_hbm, dst_hbm, a2_hbm, zeros_hbm,
                  out_sig, out_h,
                  idx_s, idx_d, buf, a2buf, acc, sem0, sem1):
    cid = lax.axis_index("c")
    sid = lax.axis_index("s")

    @pl.when(sid == 0)
    def _():
        pltpu.sync_copy(zeros_hbm, acc)

    plsc.subcore_barrier()

    def chunk(i, carry):
        start = (i * _NS + sid) * _C
        pltpu.sync_copy(dst_hbm.at[pl.ds(start, _C)], idx_d)
        pltpu.sync_copy(sig_hbm.at[pl.ds(start, _C)], buf)

        @pl.when(cid == 1)
        def _():
            pltpu.sync_copy(src_hbm.at[pl.ds(start, _C)], idx_s)
            pltpu.async_copy(a2_hbm.at[idx_s], a2buf, sem1).wait()

            def row(r, c2):
                for j in range(_VPR):
                    sl = pl.ds(j * _LANES, _LANES)
                    buf[r, sl] = buf[r, sl] * a2buf[r, sl]
                return c2
            lax.fori_loop(0, _C, row, 0)

        pltpu.sync_copy(buf, acc.at[idx_d], add=True)
        return carry

    lax.fori_loop(0, _NB // _NS, chunk, 0)

    plsc.subcore_barrier()

    @pl.when((sid == 0) & (cid == 0))
    def _():
        pltpu.sync_copy(acc, out_sig)

    @pl.when((sid == 0) & (cid == 1))
    def _():
        pltpu.sync_copy(acc, out_h)


def _sc_mesh():
    return plsc.VectorSubcoreMesh(core_axis_name="c", subcore_axis_name="s",
                                  num_cores=_NC, num_subcores=_NS)


def _gather_sum(b1h, b2h, src, dst):
    return pl.kernel(
        _gather_sum_body,
        out_type=jax.ShapeDtypeStruct((E, D), jnp.float32),
        mesh=_sc_mesh(),
        scratch_types=[
            pltpu.VMEM((_C,), jnp.int32),
            pltpu.VMEM((_C,), jnp.int32),
            pltpu.VMEM((_C, D), jnp.float32),
            pltpu.VMEM((_C, D), jnp.float32),
            pltpu.SemaphoreType.DMA,
            pltpu.SemaphoreType.DMA,
        ],
    )(b1h, b2h, src, dst)


def _scatter_sums(sig, src, dst, a2h, zeros_nd):
    return pl.kernel(
        _scatter_body,
        out_type=(jax.ShapeDtypeStruct((N, D), jnp.float32),
                  jax.ShapeDtypeStruct((N, D), jnp.float32)),
        mesh=_sc_mesh(),
        scratch_types=[
            pltpu.VMEM((_C,), jnp.int32),
            pltpu.VMEM((_C,), jnp.int32),
            pltpu.VMEM((_C, D), jnp.float32),
            pltpu.VMEM((_C, D), jnp.float32),
            pltpu.VMEM_SHARED((N, D), jnp.float32),
            pltpu.SemaphoreType.DMA,
            pltpu.SemaphoreType.DMA,
        ],
    )(sig, src, dst, a2h, zeros_nd)


def kernel(h, edge_index, e, W_A1, b_A1, W_A2, b_A2, W_B1, b_B1,
           W_B2, b_B2, W_B3, b_B3, g_h, be_h, g_e, be_e):
    src = edge_index[0]
    dst = edge_index[1]

    zpad = jnp.zeros((4, D), jnp.float32)
    nbias = jnp.concatenate([b_A1[None], b_A2[None], b_B1[None], b_B2[None],
                             zpad], axis=0)
    evec = jnp.concatenate([b_B3[None], g_e[None], be_e[None], zpad,
                            jnp.zeros((1, D), jnp.float32)], axis=0)
    hvec = jnp.concatenate([g_h[None], be_h[None], zpad,
                            jnp.zeros((2, D), jnp.float32)], axis=0)

    a1h, a2h, b1h, b2h = _mm4(h, W_A1.T, W_A2.T, W_B1.T, W_B2.T, nbias)
    bsum = _gather_sum(b1h, b2h, src, dst)
    e_ji, sig = _edge_fused(e, bsum, W_B3.T, evec)
    zeros_nd = jnp.zeros((N, D), jnp.float32)
    sum_sig, sum_h = _scatter_sums(sig, src, dst, a2h, zeros_nd)
    h_out = _node_final(a1h, sum_h, sum_sig, h, hvec)
    return (h_out, e_ji)


# speedup vs baseline: 4.5211x; 1.7269x over previous
import functools

import jax
import jax.numpy as jnp
from jax import lax
from jax.experimental import pallas as pl
from jax.experimental.pallas import tpu as pltpu
from jax.experimental.pallas import tpu_sc as plsc

N = 10000
E = 320000
D = 128

_NC = 2
_NS = 16
_NW = _NC * _NS

_C = 80
_NB = E // _C
_LANES = 16
_VPR = D // _LANES

_BN = 2000
_BE = 4000


def _mm4_body(h_ref, w1_ref, w2_ref, w3_ref, w4_ref, b_ref,
              o1_ref, o2_ref, o3_ref, o4_ref):
    hb = h_ref[...]
    o1_ref[...] = jnp.dot(hb, w1_ref[...], preferred_element_type=jnp.float32) + b_ref[0:1, :]
    o2_ref[...] = jnp.dot(hb, w2_ref[...], preferred_element_type=jnp.float32) + b_ref[1:2, :]
    o3_ref[...] = jnp.dot(hb, w3_ref[...], preferred_element_type=jnp.float32) + b_ref[2:3, :]
    o4_ref[...] = jnp.dot(hb, w4_ref[...], preferred_element_type=jnp.float32) + b_ref[3:4, :]


def _edge_body(e_ref, bsum_ref, wt_ref, vec_ref, out_ref, sig_ref):
    eb = e_ref[...]
    x = jnp.dot(eb, wt_ref[...], preferred_element_type=jnp.float32)
    x = x + bsum_ref[...] + vec_ref[0:1, :]
    mu = jnp.mean(x, axis=1, keepdims=True)
    xc = x - mu
    var = jnp.mean(xc * xc, axis=1, keepdims=True)
    y = xc * lax.rsqrt(var + 1e-5)
    y = y * vec_ref[1:2, :] + vec_ref[2:3, :]
    e_ji = jnp.maximum(y, 0.0) + eb
    out_ref[...] = e_ji
    sig_ref[...] = jax.nn.sigmoid(e_ji)


def _final_body(a1_ref, sh_ref, ss_ref, h_ref, vec_ref, out_ref):
    x = a1_ref[...] + sh_ref[...] / (ss_ref[...] + 1e-6)
    mu = jnp.mean(x, axis=1, keepdims=True)
    xc = x - mu
    var = jnp.mean(xc * xc, axis=1, keepdims=True)
    y = xc * lax.rsqrt(var + 1e-5)
    y = y * vec_ref[0:1, :] + vec_ref[1:2, :]
    out_ref[...] = jnp.maximum(y, 0.0) + h_ref[...]


def _node_spec(i):
    return (i, 0)


def _rep_spec(i):
    return (0, 0)


def _mm4(h, wt1, wt2, wt3, wt4, nbias):
    grid = (N // _BN,)
    blk = pl.BlockSpec((_BN, D), _node_spec)
    wspec = pl.BlockSpec((D, D), _rep_spec)
    return pl.pallas_call(
        _mm4_body,
        grid=grid,
        in_specs=[blk, wspec, wspec, wspec, wspec,
                  pl.BlockSpec((8, D), _rep_spec)],
        out_specs=[blk, blk, blk, blk],
        out_shape=[jax.ShapeDtypeStruct((N, D), jnp.float32)] * 4,
    )(h, wt1, wt2, wt3, wt4, nbias)


def _edge_fused(e, bsum, wt3, evec):
    grid = (E // _BE,)
    blk = pl.BlockSpec((_BE, D), _node_spec)
    return pl.pallas_call(
        _edge_body,
        grid=grid,
        in_specs=[blk, blk, pl.BlockSpec((D, D), _rep_spec),
                  pl.BlockSpec((8, D), _rep_spec)],
        out_specs=[blk, blk],
        out_shape=[jax.ShapeDtypeStruct((E, D), jnp.float32)] * 2,
    )(e, bsum, wt3, evec)


def _node_final(a1h, sum_h, sum_sig, h, hvec):
    grid = (N // _BN,)
    blk = pl.BlockSpec((_BN, D), _node_spec)
    return pl.pallas_call(
        _final_body,
        grid=grid,
        in_specs=[blk, blk, blk, blk, pl.BlockSpec((8, D), _rep_spec)],
        out_specs=blk,
        out_shape=jax.ShapeDtypeStruct((N, D), jnp.float32),
    )(a1h, sum_h, sum_sig, h, hvec)


def _gather_sum_body(b1_hbm, b2_hbm, idx2_hbm, out_hbm,
                     ib0, ib1, g1_0, g1_1, g2_0, g2_1,
                     sg0, sg1, so0, so1):
    cid = lax.axis_index("c")
    sid = lax.axis_index("s")
    w = sid * _NC + cid
    ib = (ib0, ib1)
    g1 = (g1_0, g1_1)
    g2 = (g2_0, g2_1)
    sg = (sg0, sg1)
    so = (so0, so1)
    nch = _NB // _NW

    def issue(t, s):
        blk = t * _NW + w
        pltpu.sync_copy(idx2_hbm.at[blk], ib[s])
        pltpu.async_copy(b1_hbm.at[ib[s].at[0]], g1[s], sg[s])
        pltpu.async_copy(b2_hbm.at[ib[s].at[1]], g2[s], sg[s])

    def wait_gather(s):
        pltpu.make_async_copy(b1_hbm.at[ib[s].at[0]], g1[s], sg[s]).wait()
        pltpu.make_async_copy(b2_hbm.at[ib[s].at[1]], g2[s], sg[s]).wait()

    def add_store(t, s):
        def row(r, c2):
            for j in range(_VPR):
                sl = pl.ds(j * _LANES, _LANES)
                g1[s][r, sl] = g1[s][r, sl] + g2[s][r, sl]
            return c2
        lax.fori_loop(0, _C, row, 0)
        blk = t * _NW + w
        pltpu.async_copy(g1[s], out_hbm.at[pl.ds(blk * _C, _C)], so[s])

    def wait_out(s):
        pltpu.make_async_copy(g1[s], out_hbm.at[pl.ds(0, _C)], so[s]).wait()

    issue(0, 0)
    wait_gather(0)
    issue(1, 1)
    add_store(0, 0)

    def pair(u, carry):
        t1 = 2 * u + 1
        wait_gather(1)
        wait_out(0)
        issue(t1 + 1, 0)
        add_store(t1, 1)
        wait_gather(0)
        wait_out(1)
        issue(t1 + 2, 1)
        add_store(t1 + 1, 0)
        return carry

    lax.fori_loop(0, (nch - 3) // 2, pair, 0)

    wait_gather(1)
    wait_out(0)
    issue(nch - 1, 0)
    add_store(nch - 2, 1)
    wait_gather(0)
    wait_out(1)
    add_store(nch - 1, 0)
    wait_out(0)


def _scatter_body(sig_hbm, idx2_hbm, a2_hbm, zeros_hbm,
                  out_sig, out_h,
                  ib0, ib1, sb0, sb1, ab0, ab1, acc,
                  sl0, sl1, sa0, sa1, sc0, sc1):
    cid = lax.axis_index("c")
    sid = lax.axis_index("s")
    ib = (ib0, ib1)
    sb = (sb0, sb1)
    ab = (ab0, ab1)
    slm = (sl0, sl1)
    sam = (sa0, sa1)
    scm = (sc0, sc1)
    nch = _NB // _NS

    @pl.when(sid == 0)
    def _():
        pltpu.sync_copy(zeros_hbm, acc)

    plsc.subcore_barrier()

    def issue(t, s):
        blk = t * _NS + sid
        pltpu.sync_copy(idx2_hbm.at[blk], ib[s])
        pltpu.async_copy(sig_hbm.at[pl.ds(blk * _C, _C)], sb[s], slm[s])

        @pl.when(cid == 1)
        def _():
            pltpu.async_copy(a2_hbm.at[ib[s].at[0]], ab[s], sam[s])

    def proc(t, s):
        pltpu.make_async_copy(sig_hbm.at[pl.ds(0, _C)], sb[s], slm[s]).wait()

        @pl.when(cid == 1)
        def _():
            pltpu.make_async_copy(a2_hbm.at[ib[s].at[0]], ab[s], sam[s]).wait()

            def row(r, c2):
                for j in range(_VPR):
                    sl = pl.ds(j * _LANES, _LANES)
                    sb[s][r, sl] = sb[s][r, sl] * ab[s][r, sl]
                return c2
            lax.fori_loop(0, _C, row, 0)

        pltpu.async_copy(sb[s], acc.at[ib[s].at[1]], scm[s], add=True)

    def wait_scat(s):
        pltpu.make_async_copy(sb[s], acc.at[ib[s].at[1]], scm[s]).wait()

    issue(0, 0)
    issue(1, 1)
    proc(0, 0)

    def pair(u, carry):
        t1 = 2 * u + 1
        wait_scat(0)
        issue(t1 + 1, 0)
        proc(t1, 1)
        wait_scat(1)
        issue(t1 + 2, 1)
        proc(t1 + 1, 0)
        return carry

    lax.fori_loop(0, (nch - 2) // 2, pair, 0)

    proc(nch - 1, 1)
    wait_scat(0)
    wait_scat(1)

    plsc.subcore_barrier()

    @pl.when((sid == 0) & (cid == 0))
    def _():
        pltpu.sync_copy(acc, out_sig)

    @pl.when((sid == 0) & (cid == 1))
    def _():
        pltpu.sync_copy(acc, out_h)


def _sc_mesh():
    return plsc.VectorSubcoreMesh(core_axis_name="c", subcore_axis_name="s",
                                  num_cores=_NC, num_subcores=_NS)


def _gather_sum(b1h, b2h, idx2):
    return pl.kernel(
        _gather_sum_body,
        out_type=jax.ShapeDtypeStruct((E, D), jnp.float32),
        mesh=_sc_mesh(),
        scratch_types=[
            pltpu.VMEM((2, _C), jnp.int32),
            pltpu.VMEM((2, _C), jnp.int32),
            pltpu.VMEM((_C, D), jnp.float32),
            pltpu.VMEM((_C, D), jnp.float32),
            pltpu.VMEM((_C, D), jnp.float32),
            pltpu.VMEM((_C, D), jnp.float32),
            pltpu.SemaphoreType.DMA,
            pltpu.SemaphoreType.DMA,
            pltpu.SemaphoreType.DMA,
            pltpu.SemaphoreType.DMA,
        ],
    )(b1h, b2h, idx2)


def _scatter_sums(sig, idx2, a2h, zeros_nd):
    return pl.kernel(
        _scatter_body,
        out_type=(jax.ShapeDtypeStruct((N, D), jnp.float32),
                  jax.ShapeDtypeStruct((N, D), jnp.float32)),
        mesh=_sc_mesh(),
        scratch_types=[
            pltpu.VMEM((2, _C), jnp.int32),
            pltpu.VMEM((2, _C), jnp.int32),
            pltpu.VMEM((_C, D), jnp.float32),
            pltpu.VMEM((_C, D), jnp.float32),
            pltpu.VMEM((_C, D), jnp.float32),
            pltpu.VMEM((_C, D), jnp.float32),
            pltpu.VMEM_SHARED((N, D), jnp.float32),
            pltpu.SemaphoreType.DMA,
            pltpu.SemaphoreType.DMA,
            pltpu.SemaphoreType.DMA,
            pltpu.SemaphoreType.DMA,
            pltpu.SemaphoreType.DMA,
            pltpu.SemaphoreType.DMA,
        ],
    )(sig, idx2, a2h, zeros_nd)


def kernel(h, edge_index, e, W_A1, b_A1, W_A2, b_A2, W_B1, b_B1,
           W_B2, b_B2, W_B3, b_B3, g_h, be_h, g_e, be_e):
    src = edge_index[0]
    dst = edge_index[1]

    zpad = jnp.zeros((4, D), jnp.float32)
    nbias = jnp.concatenate([b_A1[None], b_A2[None], b_B1[None], b_B2[None],
                             zpad], axis=0)
    evec = jnp.concatenate([b_B3[None], g_e[None], be_e[None], zpad,
                            jnp.zeros((1, D), jnp.float32)], axis=0)
    hvec = jnp.concatenate([g_h[None], be_h[None], zpad,
                            jnp.zeros((2, D), jnp.float32)], axis=0)

    idx2 = jnp.stack([src.reshape(_NB, _C), dst.reshape(_NB, _C)], axis=1)

    a1h, a2h, b1h, b2h = _mm4(h, W_A1.T, W_A2.T, W_B1.T, W_B2.T, nbias)
    bsum = _gather_sum(b1h, b2h, idx2)
    e_ji, sig = _edge_fused(e, bsum, W_B3.T, evec)
    zeros_nd = jnp.zeros((N, D), jnp.float32)
    sum_sig, sum_h = _scatter_sums(sig, idx2, a2h, zeros_nd)
    h_out = _node_final(a1h, sum_h, sum_sig, h, hvec)
    return (h_out, e_ji)


# speedup vs baseline: 5.1988x; 1.1499x over previous
import functools

import jax
import jax.numpy as jnp
from jax import lax
from jax.experimental import pallas as pl
from jax.experimental.pallas import tpu as pltpu
from jax.experimental.pallas import tpu_sc as plsc

N = 10000
E = 320000
D = 128

_NC = 2
_NS = 16
_NW = _NC * _NS

_C = 80
_NB = E // _C
_LANES = 16
_VPR = D // _LANES

_BN = 2000
_BE = 4000

_NB1 = 1600
_E1 = _NB1 * _C
_EB1 = _E1 // _BE


def _mm4_body(h_ref, w1_ref, w2_ref, w3_ref, w4_ref, b_ref,
              o1_ref, o2_ref, o3_ref, o4_ref):
    hb = h_ref[...]
    o1_ref[...] = jnp.dot(hb, w1_ref[...], preferred_element_type=jnp.float32) + b_ref[0:1, :]
    o2_ref[...] = jnp.dot(hb, w2_ref[...], preferred_element_type=jnp.float32) + b_ref[1:2, :]
    o3_ref[...] = jnp.dot(hb, w3_ref[...], preferred_element_type=jnp.float32) + b_ref[2:3, :]
    o4_ref[...] = jnp.dot(hb, w4_ref[...], preferred_element_type=jnp.float32) + b_ref[3:4, :]


def _edge_body(e_ref, bsum_ref, wt_ref, vec_ref, out_ref, sig_ref):
    eb = e_ref[...]
    x = jnp.dot(eb, wt_ref[...], preferred_element_type=jnp.float32)
    x = x + bsum_ref[...] + vec_ref[0:1, :]
    mu = jnp.mean(x, axis=1, keepdims=True)
    xc = x - mu
    var = jnp.mean(xc * xc, axis=1, keepdims=True)
    y = xc * lax.rsqrt(var + 1e-5)
    y = y * vec_ref[1:2, :] + vec_ref[2:3, :]
    e_ji = jnp.maximum(y, 0.0) + eb
    out_ref[...] = e_ji
    sig_ref[...] = jax.nn.sigmoid(e_ji)


def _final_body(a1_ref, sh1_ref, sh2_ref, ss1_ref, ss2_ref, h_ref,
                vec_ref, out_ref):
    sh = sh1_ref[...] + sh2_ref[...]
    ss = ss1_ref[...] + ss2_ref[...]
    x = a1_ref[...] + sh / (ss + 1e-6)
    mu = jnp.mean(x, axis=1, keepdims=True)
    xc = x - mu
    var = jnp.mean(xc * xc, axis=1, keepdims=True)
    y = xc * lax.rsqrt(var + 1e-5)
    y = y * vec_ref[0:1, :] + vec_ref[1:2, :]
    out_ref[...] = jnp.maximum(y, 0.0) + h_ref[...]


def _node_spec(i):
    return (i, 0)


def _rep_spec(i):
    return (0, 0)


def _mm4(h, wt1, wt2, wt3, wt4, nbias):
    grid = (N // _BN,)
    blk = pl.BlockSpec((_BN, D), _node_spec)
    wspec = pl.BlockSpec((D, D), _rep_spec)
    return pl.pallas_call(
        _mm4_body,
        grid=grid,
        in_specs=[blk, wspec, wspec, wspec, wspec,
                  pl.BlockSpec((8, D), _rep_spec)],
        out_specs=[blk, blk, blk, blk],
        out_shape=[jax.ShapeDtypeStruct((N, D), jnp.float32)] * 4,
    )(h, wt1, wt2, wt3, wt4, nbias)


def _edge_fused1(e, bsum1, wt3, evec):
    blk = pl.BlockSpec((_BE, D), _node_spec)
    return pl.pallas_call(
        _edge_body,
        grid=(_EB1,),
        in_specs=[blk, blk, pl.BlockSpec((D, D), _rep_spec),
                  pl.BlockSpec((8, D), _rep_spec)],
        out_specs=[blk, blk],
        out_shape=[jax.ShapeDtypeStruct((E, D), jnp.float32),
                   jax.ShapeDtypeStruct((_E1, D), jnp.float32)],
    )(e, bsum1, wt3, evec)


def _edge_body2(e_ref, bsum_ref, wt_ref, vec_ref, buf_ref, out_ref, sig_ref):
    _edge_body(e_ref, bsum_ref, wt_ref, vec_ref, out_ref, sig_ref)


def _edge_fused2(e, bsum2, wt3, evec, eji_buf):
    def off_spec(i):
        return (i + _EB1, 0)
    blk = pl.BlockSpec((_BE, D), _node_spec)
    oblk = pl.BlockSpec((_BE, D), off_spec)
    return pl.pallas_call(
        _edge_body2,
        grid=(E // _BE - _EB1,),
        in_specs=[oblk, blk, pl.BlockSpec((D, D), _rep_spec),
                  pl.BlockSpec((8, D), _rep_spec),
                  pl.BlockSpec(memory_space=pl.ANY)],
        out_specs=[oblk, blk],
        out_shape=[jax.ShapeDtypeStruct((E, D), jnp.float32),
                   jax.ShapeDtypeStruct((E - _E1, D), jnp.float32)],
        input_output_aliases={4: 0},
    )(e, bsum2, wt3, evec, eji_buf)


def _node_final(a1h, sh1, sh2, ss1, ss2, h, hvec):
    grid = (N // _BN,)
    blk = pl.BlockSpec((_BN, D), _node_spec)
    return pl.pallas_call(
        _final_body,
        grid=grid,
        in_specs=[blk, blk, blk, blk, blk, blk,
                  pl.BlockSpec((8, D), _rep_spec)],
        out_specs=blk,
        out_shape=jax.ShapeDtypeStruct((N, D), jnp.float32),
    )(a1h, sh1, sh2, ss1, ss2, h, hvec)


def _pipe(nch, issue, proc, wait_slot):
    issue(0, 0)
    issue(1, 1)
    proc(0, 0)

    def pair(u, carry):
        t1 = 2 * u + 1
        wait_slot(0)
        issue(t1 + 1, 0)
        proc(t1, 1)
        wait_slot(1)
        issue(t1 + 2, 1)
        proc(t1 + 1, 0)
        return carry

    if nch % 2 == 0:
        lax.fori_loop(0, (nch - 2) // 2, pair, 0)
        proc(nch - 1, 1)
        wait_slot(0)
        wait_slot(1)
    else:
        lax.fori_loop(0, (nch - 3) // 2, pair, 0)
        wait_slot(0)
        issue(nch - 1, 0)
        proc(nch - 2, 1)
        wait_slot(1)
        proc(nch - 1, 0)
        wait_slot(0)


def _gather_sum_body(b1_hbm, b2_hbm, idx2_hbm, out_hbm,
                     ib0, ib1, g1_0, g1_1, g2_0, g2_1,
                     sg0, sg1, so0, so1, *, base, per):
    cid = lax.axis_index("c")
    sid = lax.axis_index("s")
    w = sid * _NC + cid
    ib = (ib0, ib1)
    g1 = (g1_0, g1_1)
    g2 = (g2_0, g2_1)
    sg = (sg0, sg1)
    so = (so0, so1)
    loc0 = w * per

    def issue(t, s):
        loc = loc0 + t
        pltpu.sync_copy(idx2_hbm.at[base + loc], ib[s])
        pltpu.async_copy(b1_hbm.at[ib[s].at[0]], g1[s], sg[s])
        pltpu.async_copy(b2_hbm.at[ib[s].at[1]], g2[s], sg[s])

    def proc(t, s):
        pltpu.make_async_copy(b1_hbm.at[ib[s].at[0]], g1[s], sg[s]).wait()
        pltpu.make_async_copy(b2_hbm.at[ib[s].at[1]], g2[s], sg[s]).wait()

        def row(r, c2):
            for j in range(_VPR):
                sl = pl.ds(j * _LANES, _LANES)
                g1[s][r, sl] = g1[s][r, sl] + g2[s][r, sl]
            return c2
        lax.fori_loop(0, _C, row, 0)
        loc = loc0 + t
        pltpu.async_copy(g1[s], out_hbm.at[pl.ds(loc * _C, _C)], so[s])

    def wait_slot(s):
        pltpu.make_async_copy(g1[s], out_hbm.at[pl.ds(0, _C)], so[s]).wait()

    _pipe(per, issue, proc, wait_slot)


def _scatter_body(sig_hbm, idx2_hbm, a2_hbm, zeros_hbm,
                  out_sig, out_h,
                  ib0, ib1, sb0, sb1, ab0, ab1, acc,
                  sl0, sl1, sa0, sa1, sc0, sc1, *, base, per):
    cid = lax.axis_index("c")
    sid = lax.axis_index("s")
    ib = (ib0, ib1)
    sb = (sb0, sb1)
    ab = (ab0, ab1)
    slm = (sl0, sl1)
    sam = (sa0, sa1)
    scm = (sc0, sc1)
    loc0 = sid * per

    @pl.when(sid == 0)
    def _():
        pltpu.sync_copy(zeros_hbm, acc)

    plsc.subcore_barrier()

    def issue(t, s):
        loc = loc0 + t
        pltpu.sync_copy(idx2_hbm.at[base + loc], ib[s])
        pltpu.async_copy(sig_hbm.at[pl.ds(loc * _C, _C)], sb[s], slm[s])

        @pl.when(cid == 1)
        def _():
            pltpu.async_copy(a2_hbm.at[ib[s].at[0]], ab[s], sam[s])

    def proc(t, s):
        pltpu.make_async_copy(sig_hbm.at[pl.ds(0, _C)], sb[s], slm[s]).wait()

        @pl.when(cid == 1)
        def _():
            pltpu.make_async_copy(a2_hbm.at[ib[s].at[0]], ab[s], sam[s]).wait()

            def row(r, c2):
                for j in range(_VPR):
                    sl = pl.ds(j * _LANES, _LANES)
                    sb[s][r, sl] = sb[s][r, sl] * ab[s][r, sl]
                return c2
            lax.fori_loop(0, _C, row, 0)

        pltpu.async_copy(sb[s], acc.at[ib[s].at[1]], scm[s], add=True)

    def wait_slot(s):
        pltpu.make_async_copy(sb[s], acc.at[ib[s].at[1]], scm[s]).wait()

    _pipe(per, issue, proc, wait_slot)

    plsc.subcore_barrier()

    @pl.when((sid == 0) & (cid == 0))
    def _():
        pltpu.sync_copy(acc, out_sig)

    @pl.when((sid == 0) & (cid == 1))
    def _():
        pltpu.sync_copy(acc, out_h)


def _sc_mesh():
    return plsc.VectorSubcoreMesh(core_axis_name="c", subcore_axis_name="s",
                                  num_cores=_NC, num_subcores=_NS)


def _gather_sum(b1h, b2h, idx2, base, nchunks):
    body = functools.partial(_gather_sum_body, base=base, per=nchunks // _NW)
    return pl.kernel(
        body,
        out_type=jax.ShapeDtypeStruct((nchunks * _C, D), jnp.float32),
        mesh=_sc_mesh(),
        scratch_types=[
            pltpu.VMEM((2, _C), jnp.int32),
            pltpu.VMEM((2, _C), jnp.int32),
            pltpu.VMEM((_C, D), jnp.float32),
            pltpu.VMEM((_C, D), jnp.float32),
            pltpu.VMEM((_C, D), jnp.float32),
            pltpu.VMEM((_C, D), jnp.float32),
            pltpu.SemaphoreType.DMA,
            pltpu.SemaphoreType.DMA,
            pltpu.SemaphoreType.DMA,
            pltpu.SemaphoreType.DMA,
        ],
    )(b1h, b2h, idx2)


def _scatter_sums(sig, idx2, a2h, zeros_nd, base, nchunks):
    body = functools.partial(_scatter_body, base=base, per=nchunks // _NS)
    return pl.kernel(
        body,
        out_type=(jax.ShapeDtypeStruct((N, D), jnp.float32),
                  jax.ShapeDtypeStruct((N, D), jnp.float32)),
        mesh=_sc_mesh(),
        scratch_types=[
            pltpu.VMEM((2, _C), jnp.int32),
            pltpu.VMEM((2, _C), jnp.int32),
            pltpu.VMEM((_C, D), jnp.float32),
            pltpu.VMEM((_C, D), jnp.float32),
            pltpu.VMEM((_C, D), jnp.float32),
            pltpu.VMEM((_C, D), jnp.float32),
            pltpu.VMEM_SHARED((N, D), jnp.float32),
            pltpu.SemaphoreType.DMA,
            pltpu.SemaphoreType.DMA,
            pltpu.SemaphoreType.DMA,
            pltpu.SemaphoreType.DMA,
            pltpu.SemaphoreType.DMA,
            pltpu.SemaphoreType.DMA,
        ],
    )(sig, idx2, a2h, zeros_nd)


def kernel(h, edge_index, e, W_A1, b_A1, W_A2, b_A2, W_B1, b_B1,
           W_B2, b_B2, W_B3, b_B3, g_h, be_h, g_e, be_e):
    src = edge_index[0]
    dst = edge_index[1]

    zpad = jnp.zeros((4, D), jnp.float32)
    nbias = jnp.concatenate([b_A1[None], b_A2[None], b_B1[None], b_B2[None],
                             zpad], axis=0)
    evec = jnp.concatenate([b_B3[None], g_e[None], be_e[None], zpad,
                            jnp.zeros((1, D), jnp.float32)], axis=0)
    hvec = jnp.concatenate([g_h[None], be_h[None], zpad,
                            jnp.zeros((2, D), jnp.float32)], axis=0)

    idx2 = jnp.stack([src.reshape(_NB, _C), dst.reshape(_NB, _C)], axis=1)
    wt3 = W_B3.T
    zeros_nd = jnp.zeros((N, D), jnp.float32)

    a1h, a2h, b1h, b2h = _mm4(h, W_A1.T, W_A2.T, W_B1.T, W_B2.T, nbias)

    bsum1 = _gather_sum(b1h, b2h, idx2, 0, _NB1)
    bsum2 = _gather_sum(b1h, b2h, idx2, _NB1, _NB - _NB1)
    eji_partial, sig1 = _edge_fused1(e, bsum1, wt3, evec)
    ss1, sh1 = _scatter_sums(sig1, idx2, a2h, zeros_nd, 0, _NB1)
    e_ji, sig2 = _edge_fused2(e, bsum2, wt3, evec, eji_partial)
    ss2, sh2 = _scatter_sums(sig2, idx2, a2h, zeros_nd, _NB1, _NB - _NB1)

    h_out = _node_final(a1h, sh1, sh2, ss1, ss2, h, hvec)
    return (h_out, e_ji)


# speedup vs baseline: 5.2017x; 1.0006x over previous
import functools

import jax
import jax.numpy as jnp
from jax import lax
from jax.experimental import pallas as pl
from jax.experimental.pallas import tpu as pltpu
from jax.experimental.pallas import tpu_sc as plsc

N = 10000
E = 320000
D = 128

_NC = 2
_NS = 16
_NW = _NC * _NS

_C = 80
_NB = E // _C
_LANES = 16
_VPR = D // _LANES

_BN = 2000
_BE = 4000

_NB1 = 1600
_E1 = _NB1 * _C
_EB1 = _E1 // _BE


def _mm4_body(h_ref, w1_ref, w2_ref, w3_ref, w4_ref, b_ref,
              o1_ref, o2_ref, o3_ref, o4_ref):
    hb = h_ref[...]
    o1_ref[...] = jnp.dot(hb, w1_ref[...], preferred_element_type=jnp.float32) + b_ref[0:1, :]
    o2_ref[...] = jnp.dot(hb, w2_ref[...], preferred_element_type=jnp.float32) + b_ref[1:2, :]
    o3_ref[...] = jnp.dot(hb, w3_ref[...], preferred_element_type=jnp.float32) + b_ref[2:3, :]
    o4_ref[...] = jnp.dot(hb, w4_ref[...], preferred_element_type=jnp.float32) + b_ref[3:4, :]


def _edge_body(e_ref, bsum_ref, wt_ref, vec_ref, out_ref, sig_ref):
    eb = e_ref[...]
    x = jnp.dot(eb, wt_ref[...], preferred_element_type=jnp.float32)
    x = x + bsum_ref[...] + vec_ref[0:1, :]
    mu = jnp.mean(x, axis=1, keepdims=True)
    xc = x - mu
    var = jnp.mean(xc * xc, axis=1, keepdims=True)
    y = xc * lax.rsqrt(var + 1e-5)
    y = y * vec_ref[1:2, :] + vec_ref[2:3, :]
    e_ji = jnp.maximum(y, 0.0) + eb
    out_ref[...] = e_ji
    sig_ref[...] = jax.nn.sigmoid(e_ji)


def _final_body(a1_ref, sh1_ref, sh2_ref, ss1_ref, ss2_ref, h_ref,
                vec_ref, out_ref):
    sh = sh1_ref[...] + sh2_ref[...]
    ss = ss1_ref[...] + ss2_ref[...]
    x = a1_ref[...] + sh / (ss + 1e-6)
    mu = jnp.mean(x, axis=1, keepdims=True)
    xc = x - mu
    var = jnp.mean(xc * xc, axis=1, keepdims=True)
    y = xc * lax.rsqrt(var + 1e-5)
    y = y * vec_ref[0:1, :] + vec_ref[1:2, :]
    out_ref[...] = jnp.maximum(y, 0.0) + h_ref[...]


def _node_spec(i):
    return (i, 0)


def _rep_spec(i):
    return (0, 0)


def _mm4(h, wt1, wt2, wt3, wt4, nbias):
    grid = (N // _BN,)
    blk = pl.BlockSpec((_BN, D), _node_spec)
    wspec = pl.BlockSpec((D, D), _rep_spec)
    return pl.pallas_call(
        _mm4_body,
        grid=grid,
        in_specs=[blk, wspec, wspec, wspec, wspec,
                  pl.BlockSpec((8, D), _rep_spec)],
        out_specs=[blk, blk, blk, blk],
        out_shape=[jax.ShapeDtypeStruct((N, D), jnp.float32)] * 4,
    )(h, wt1, wt2, wt3, wt4, nbias)


def _edge_fused1(e, bsum1, wt3, evec):
    blk = pl.BlockSpec((_BE, D), _node_spec)
    return pl.pallas_call(
        _edge_body,
        grid=(_EB1,),
        in_specs=[blk, blk, pl.BlockSpec((D, D), _rep_spec),
                  pl.BlockSpec((8, D), _rep_spec)],
        out_specs=[blk, blk],
        out_shape=[jax.ShapeDtypeStruct((E, D), jnp.float32),
                   jax.ShapeDtypeStruct((_E1, D), jnp.float32)],
    )(e, bsum1, wt3, evec)


def _edge_body2(e_ref, bsum_ref, wt_ref, vec_ref, buf_ref, out_ref, sig_ref):
    _edge_body(e_ref, bsum_ref, wt_ref, vec_ref, out_ref, sig_ref)


def _edge_fused2(e, bsum2, wt3, evec, eji_buf):
    def off_spec(i):
        return (i + _EB1, 0)
    blk = pl.BlockSpec((_BE, D), _node_spec)
    oblk = pl.BlockSpec((_BE, D), off_spec)
    return pl.pallas_call(
        _edge_body2,
        grid=(E // _BE - _EB1,),
        in_specs=[oblk, blk, pl.BlockSpec((D, D), _rep_spec),
                  pl.BlockSpec((8, D), _rep_spec),
                  pl.BlockSpec(memory_space=pl.ANY)],
        out_specs=[oblk, blk],
        out_shape=[jax.ShapeDtypeStruct((E, D), jnp.float32),
                   jax.ShapeDtypeStruct((E - _E1, D), jnp.float32)],
        input_output_aliases={4: 0},
    )(e, bsum2, wt3, evec, eji_buf)


def _node_final(a1h, sh1, sh2, ss1, ss2, h, hvec):
    grid = (N // _BN,)
    blk = pl.BlockSpec((_BN, D), _node_spec)
    return pl.pallas_call(
        _final_body,
        grid=grid,
        in_specs=[blk, blk, blk, blk, blk, blk,
                  pl.BlockSpec((8, D), _rep_spec)],
        out_specs=blk,
        out_shape=jax.ShapeDtypeStruct((N, D), jnp.float32),
    )(a1h, sh1, sh2, ss1, ss2, h, hvec)


def _pipe(nch, issue, proc, wait_slot):
    issue(0, 0)
    issue(1, 1)
    proc(0, 0)

    def pair(u, carry):
        t1 = 2 * u + 1
        wait_slot(0)
        issue(t1 + 1, 0)
        proc(t1, 1)
        wait_slot(1)
        issue(t1 + 2, 1)
        proc(t1 + 1, 0)
        return carry

    if nch % 2 == 0:
        lax.fori_loop(0, (nch - 2) // 2, pair, 0)
        proc(nch - 1, 1)
        wait_slot(0)
        wait_slot(1)
    else:
        lax.fori_loop(0, (nch - 3) // 2, pair, 0)
        wait_slot(0)
        issue(nch - 1, 0)
        proc(nch - 2, 1)
        wait_slot(1)
        proc(nch - 1, 0)
        wait_slot(0)


def _gather_sum_body(b1_hbm, b2_hbm, idx2_hbm, out_hbm,
                     ib0, ib1, g1_0, g1_1, g2_0, g2_1,
                     sg0, sg1, so0, so1, *, base, per):
    cid = lax.axis_index("c")
    sid = lax.axis_index("s")
    w = sid * _NC + cid
    ib = (ib0, ib1)
    g1 = (g1_0, g1_1)
    g2 = (g2_0, g2_1)
    sg = (sg0, sg1)
    so = (so0, so1)
    loc0 = w * per

    def issue(t, s):
        loc = loc0 + t
        pltpu.sync_copy(idx2_hbm.at[base + loc], ib[s])
        pltpu.async_copy(b1_hbm.at[ib[s].at[0]], g1[s], sg[s])
        pltpu.async_copy(b2_hbm.at[ib[s].at[1]], g2[s], sg[s])

    def proc(t, s):
        pltpu.make_async_copy(b1_hbm.at[ib[s].at[0]], g1[s], sg[s]).wait()
        pltpu.make_async_copy(b2_hbm.at[ib[s].at[1]], g2[s], sg[s]).wait()

        def row(r, c2):
            for j in range(_VPR):
                sl = pl.ds(j * _LANES, _LANES)
                g1[s][r, sl] = g1[s][r, sl] + g2[s][r, sl]
            return c2
        lax.fori_loop(0, _C, row, 0)
        loc = loc0 + t
        pltpu.async_copy(g1[s], out_hbm.at[pl.ds(loc * _C, _C)], so[s])

    def wait_slot(s):
        pltpu.make_async_copy(g1[s], out_hbm.at[pl.ds(0, _C)], so[s]).wait()

    _pipe(per, issue, proc, wait_slot)


def _scatter_body(sig_hbm, idx2_hbm, a2_hbm, zeros_hbm,
                  out_sig, out_h,
                  ib0, ib1, sb0, sb1, ab0, ab1, acc,
                  sl0, sl1, sa0, sa1, sc0, sc1, *, base, per):
    cid = lax.axis_index("c")
    sid = lax.axis_index("s")
    ib = (ib0, ib1)
    sb = (sb0, sb1)
    ab = (ab0, ab1)
    slm = (sl0, sl1)
    sam = (sa0, sa1)
    scm = (sc0, sc1)
    loc0 = sid * per

    rows = (N // _NS) // 8 * 8
    rem = N - rows * _NS
    r0 = sid * rows
    pltpu.sync_copy(zeros_hbm.at[pl.ds(r0, rows)], acc.at[pl.ds(r0, rows)])

    @pl.when(sid == 0)
    def _():
        pltpu.sync_copy(zeros_hbm.at[pl.ds(rows * _NS, rem)],
                        acc.at[pl.ds(rows * _NS, rem)])

    plsc.subcore_barrier()

    def issue(t, s):
        loc = loc0 + t
        pltpu.sync_copy(idx2_hbm.at[base + loc], ib[s])
        pltpu.async_copy(sig_hbm.at[pl.ds(loc * _C, _C)], sb[s], slm[s])

        @pl.when(cid == 1)
        def _():
            pltpu.async_copy(a2_hbm.at[ib[s].at[0]], ab[s], sam[s])

    def proc(t, s):
        pltpu.make_async_copy(sig_hbm.at[pl.ds(0, _C)], sb[s], slm[s]).wait()

        @pl.when(cid == 1)
        def _():
            pltpu.make_async_copy(a2_hbm.at[ib[s].at[0]], ab[s], sam[s]).wait()

            def row(r, c2):
                for j in range(_VPR):
                    sl = pl.ds(j * _LANES, _LANES)
                    sb[s][r, sl] = sb[s][r, sl] * ab[s][r, sl]
                return c2
            lax.fori_loop(0, _C, row, 0)

        pltpu.async_copy(sb[s], acc.at[ib[s].at[1]], scm[s], add=True)

    def wait_slot(s):
        pltpu.make_async_copy(sb[s], acc.at[ib[s].at[1]], scm[s]).wait()

    _pipe(per, issue, proc, wait_slot)

    plsc.subcore_barrier()

    @pl.when(cid == 0)
    def _():
        pltpu.sync_copy(acc.at[pl.ds(r0, rows)], out_sig.at[pl.ds(r0, rows)])

        @pl.when(sid == 0)
        def _():
            pltpu.sync_copy(acc.at[pl.ds(rows * _NS, rem)],
                            out_sig.at[pl.ds(rows * _NS, rem)])

    @pl.when(cid == 1)
    def _():
        pltpu.sync_copy(acc.at[pl.ds(r0, rows)], out_h.at[pl.ds(r0, rows)])

        @pl.when(sid == 0)
        def _():
            pltpu.sync_copy(acc.at[pl.ds(rows * _NS, rem)],
                            out_h.at[pl.ds(rows * _NS, rem)])


def _sc_mesh():
    return plsc.VectorSubcoreMesh(core_axis_name="c", subcore_axis_name="s",
                                  num_cores=_NC, num_subcores=_NS)


def _gather_sum(b1h, b2h, idx2, base, nchunks):
    body = functools.partial(_gather_sum_body, base=base, per=nchunks // _NW)
    return pl.kernel(
        body,
        out_type=jax.ShapeDtypeStruct((nchunks * _C, D), jnp.float32),
        mesh=_sc_mesh(),
        scratch_types=[
            pltpu.VMEM((2, _C), jnp.int32),
            pltpu.VMEM((2, _C), jnp.int32),
            pltpu.VMEM((_C, D), jnp.float32),
            pltpu.VMEM((_C, D), jnp.float32),
            pltpu.VMEM((_C, D), jnp.float32),
            pltpu.VMEM((_C, D), jnp.float32),
            pltpu.SemaphoreType.DMA,
            pltpu.SemaphoreType.DMA,
            pltpu.SemaphoreType.DMA,
            pltpu.SemaphoreType.DMA,
        ],
    )(b1h, b2h, idx2)


def _scatter_sums(sig, idx2, a2h, zeros_nd, base, nchunks):
    body = functools.partial(_scatter_body, base=base, per=nchunks // _NS)
    return pl.kernel(
        body,
        out_type=(jax.ShapeDtypeStruct((N, D), jnp.float32),
                  jax.ShapeDtypeStruct((N, D), jnp.float32)),
        mesh=_sc_mesh(),
        scratch_types=[
            pltpu.VMEM((2, _C), jnp.int32),
            pltpu.VMEM((2, _C), jnp.int32),
            pltpu.VMEM((_C, D), jnp.float32),
            pltpu.VMEM((_C, D), jnp.float32),
            pltpu.VMEM((_C, D), jnp.float32),
            pltpu.VMEM((_C, D), jnp.float32),
            pltpu.VMEM_SHARED((N, D), jnp.float32),
            pltpu.SemaphoreType.DMA,
            pltpu.SemaphoreType.DMA,
            pltpu.SemaphoreType.DMA,
            pltpu.SemaphoreType.DMA,
            pltpu.SemaphoreType.DMA,
            pltpu.SemaphoreType.DMA,
        ],
    )(sig, idx2, a2h, zeros_nd)


def kernel(h, edge_index, e, W_A1, b_A1, W_A2, b_A2, W_B1, b_B1,
           W_B2, b_B2, W_B3, b_B3, g_h, be_h, g_e, be_e):
    src = edge_index[0]
    dst = edge_index[1]

    zpad = jnp.zeros((4, D), jnp.float32)
    nbias = jnp.concatenate([b_A1[None], b_A2[None], b_B1[None], b_B2[None],
                             zpad], axis=0)
    evec = jnp.concatenate([b_B3[None], g_e[None], be_e[None], zpad,
                            jnp.zeros((1, D), jnp.float32)], axis=0)
    hvec = jnp.concatenate([g_h[None], be_h[None], zpad,
                            jnp.zeros((2, D), jnp.float32)], axis=0)

    idx2 = jnp.stack([src.reshape(_NB, _C), dst.reshape(_NB, _C)], axis=1)
    wt3 = W_B3.T
    zeros_nd = jnp.zeros((N, D), jnp.float32)

    a1h, a2h, b1h, b2h = _mm4(h, W_A1.T, W_A2.T, W_B1.T, W_B2.T, nbias)

    bsum1 = _gather_sum(b1h, b2h, idx2, 0, _NB1)
    bsum2 = _gather_sum(b1h, b2h, idx2, _NB1, _NB - _NB1)
    eji_partial, sig1 = _edge_fused1(e, bsum1, wt3, evec)
    ss1, sh1 = _scatter_sums(sig1, idx2, a2h, zeros_nd, 0, _NB1)
    e_ji, sig2 = _edge_fused2(e, bsum2, wt3, evec, eji_partial)
    ss2, sh2 = _scatter_sums(sig2, idx2, a2h, zeros_nd, _NB1, _NB - _NB1)

    h_out = _node_final(a1h, sh1, sh2, ss1, ss2, h, hvec)
    return (h_out, e_ji)
